# baseline (speedup 1.0000x reference)
import functools

import jax
import jax.numpy as jnp
from jax import lax
from jax.experimental import pallas as pl
from jax.experimental.pallas import tpu as pltpu
from jax.experimental.pallas import tpu_sc as plsc

NC = 2
NS = 16
LANES = 16

IDX_W = 128


@functools.lru_cache(maxsize=None)
def _build(B, L, V):
    CH = 32
    ROWS_T = B // NS
    NCH = ROWS_T // CH
    N = CH * L
    LTAIL = L - IDX_W
    assert B % NS == 0 and ROWS_T % CH == 0
    assert 0 < LTAIL <= IDX_W and LTAIL % 8 == 0 and IDX_W % 8 == 0

    inv_l = 1.0 / float(L)

    mesh = plsc.VectorSubcoreMesh(core_axis_name="c", subcore_axis_name="s")
    vm = pltpu.VMEM

    @functools.partial(
        pl.kernel,
        out_type=jax.ShapeDtypeStruct((2 * B,), jnp.float32),
        mesh=mesh,
        scratch_types=[
            vm((CH, L), jnp.int32), vm((CH, L), jnp.int32),
            vm((CH, L), jnp.float32), vm((CH, L), jnp.float32),
            vm((N,), jnp.float32), vm((N,), jnp.float32),
            vm((N,), jnp.float32), vm((N,), jnp.float32),
            vm((CH,), jnp.float32),
            pltpu.SemaphoreType.DMA, pltpu.SemaphoreType.DMA,
        ],
        compiler_params=pltpu.CompilerParams(needs_layout_passes=False),
    )
    def sc_kernel(xi_hbm, yi_hbm, xv_hbm, yv_hbm, wtf, wti, utf, uti,
                  out_hbm,
                  ix0, ix1, vv0, vv1, gtf0, gtf1, gti0, gti1,
                  outv, sem0, sem1):
        ixs, vvs = (ix0, ix1), (vv0, vv1)
        gtfs, gtis, sems = (gtf0, gtf1), (gti0, gti1), (sem0, sem1)

        cid = lax.axis_index("c")
        sid = lax.axis_index("s")
        row_lanes = lax.iota(jnp.int32, LANES) * L
        lane_iota = lax.iota(jnp.int32, LANES)

        def pipeline(idx_hbm, val_hbm, t_tf, t_ti, out_base):
            def stage(c):
                p = c % 2
                r0 = sid * ROWS_T + c * CH
                pltpu.sync_copy(idx_hbm.at[pl.ds(r0, CH), :], ixs[p])
                pltpu.sync_copy(val_hbm.at[pl.ds(r0, CH), :], vvs[p])

            def fire(c):
                p = c % 2

                def body(r, carry):
                    iw0 = ixs[p].at[r, pl.ds(0, IDX_W)]
                    iw1 = ixs[p].at[r, pl.ds(IDX_W, LTAIL)]
                    d0 = pl.ds(r * L, IDX_W)
                    d1 = pl.ds(r * L + IDX_W, LTAIL)
                    pltpu.async_copy(t_tf.at[iw0], gtfs[p].at[d0], sems[p])
                    pltpu.async_copy(t_tf.at[iw1], gtfs[p].at[d1], sems[p])
                    pltpu.async_copy(t_ti.at[iw0], gtis[p].at[d0], sems[p])
                    pltpu.async_copy(t_ti.at[iw1], gtis[p].at[d1], sems[p])
                    return carry
                lax.fori_loop(0, CH, body, 0)

            def drain(c):
                p = c % 2
                pltpu.make_async_copy(t_tf.at[pl.ds(0, N)], gtfs[p], sems[p]).wait()
                pltpu.make_async_copy(t_ti.at[pl.ds(0, N)], gtis[p], sems[p]).wait()

            def compute(c):
                p = c % 2
                for g in range(CH // LANES):
                    goff = g * LANES

                    def cbody(j, carry):
                        a_tf, a_ti = carry
                        idxv = row_lanes + (goff * L + j)
                        jcol = lane_iota * 0 + j
                        vtf = plsc.load_gather(gtfs[p], [idxv])
                        vti = plsc.load_gather(gtis[p], [idxv])
                        vq = plsc.load_gather(vvs[p], [lane_iota + goff, jcol])
                        return (a_tf + vtf, a_ti + vti * vq)
                    zero = jnp.zeros((LANES,), jnp.float32)
                    a_tf, a_ti = lax.fori_loop(0, L, cbody, (zero, zero),
                                               unroll=4)
                    outv[pl.ds(goff, LANES)] = a_tf * inv_l + a_ti
                dst = pl.ds(out_base + sid * ROWS_T + c * CH, CH)
                pltpu.sync_copy(outv, out_hbm.at[dst])

            stage(0)
            fire(0)
            for c in range(NCH):
                if c + 1 < NCH:
                    stage(c + 1)
                    fire(c + 1)
                drain(c)
                compute(c)

        @pl.when(cid == 0)
        def _():
            pipeline(xi_hbm, xv_hbm, wtf, wti, 0)

        @pl.when(cid == 1)
        def _():
            pipeline(yi_hbm, yv_hbm, utf, uti, B)

    bias = 0.0 + 0.0001 * float(L) / 10.0 + 0.0001 * float(L) / 10.0

    def fin_body(p_ref, o_ref):
        z = p_ref[0, :] + p_ref[1, :] + bias
        o_ref[...] = 1.0 / (1.0 + jnp.exp(-z))

    finisher = pl.pallas_call(
        fin_body,
        out_shape=jax.ShapeDtypeStruct((B,), jnp.float32),
    )

    def run(x_idx, y_idx, x_tfidf, y_tfidf, W_tf, U_tf, W_tfidf, U_tfidf):
        partials = sc_kernel(x_idx, y_idx, x_tfidf, y_tfidf,
                             W_tf, W_tfidf, U_tf, U_tfidf)
        return finisher(partials.reshape(2, B))

    return run


def kernel(x_idx, y_idx, x_tfidf, y_tfidf, W_tf, U_tf, W_tfidf, U_tfidf):
    B, L = x_idx.shape
    V = W_tf.shape[0]
    return _build(B, L, V)(x_idx, y_idx, x_tfidf, y_tfidf,
                           W_tf, U_tf, W_tfidf, U_tfidf)


# speedup vs baseline: 1.0650x; 1.0650x over previous
import functools

import jax
import jax.numpy as jnp
from jax import lax
from jax.experimental import pallas as pl
from jax.experimental.pallas import tpu as pltpu
from jax.experimental.pallas import tpu_sc as plsc

NC = 2
NS = 16
LANES = 16

IDX_W = 128


@functools.lru_cache(maxsize=None)
def _build(B, L, V):
    CH = 16
    ROWS_T = B // NS
    NCH = ROWS_T // CH
    N = CH * L
    LTAIL = L - IDX_W
    assert B % NS == 0 and ROWS_T % CH == 0
    assert 0 < LTAIL <= IDX_W and LTAIL % 8 == 0 and IDX_W % 8 == 0

    inv_l = 1.0 / float(L)

    mesh = plsc.VectorSubcoreMesh(core_axis_name="c", subcore_axis_name="s")
    vm = pltpu.VMEM

    @functools.partial(
        pl.kernel,
        out_type=jax.ShapeDtypeStruct((2 * B,), jnp.float32),
        mesh=mesh,
        scratch_types=[
            vm((CH, L), jnp.int32), vm((CH, L), jnp.int32),
            vm((CH, L), jnp.float32), vm((CH, L), jnp.float32),
            vm((N,), jnp.float32), vm((N,), jnp.float32),
            vm((N,), jnp.float32), vm((N,), jnp.float32),
            vm((CH,), jnp.float32),
            pltpu.SemaphoreType.DMA, pltpu.SemaphoreType.DMA,
        ],
        compiler_params=pltpu.CompilerParams(needs_layout_passes=False),
    )
    def sc_kernel(xi_hbm, yi_hbm, xv_hbm, yv_hbm, wtf, wti, utf, uti,
                  out_hbm,
                  ix0, ix1, vv0, vv1, gtf0, gtf1, gti0, gti1,
                  outv, sem0, sem1):
        ixs, vvs = (ix0, ix1), (vv0, vv1)
        gtfs, gtis, sems = (gtf0, gtf1), (gti0, gti1), (sem0, sem1)

        cid = lax.axis_index("c")
        sid = lax.axis_index("s")
        row_lanes = lax.iota(jnp.int32, LANES) * L
        lane_iota = lax.iota(jnp.int32, LANES)

        def pipeline(idx_hbm, val_hbm, t_tf, t_ti, out_base):
            def stage(c):
                p = c % 2
                r0 = sid * ROWS_T + c * CH
                pltpu.sync_copy(idx_hbm.at[pl.ds(r0, CH), :], ixs[p])
                pltpu.sync_copy(val_hbm.at[pl.ds(r0, CH), :], vvs[p])

            def fire(c):
                p = c % 2

                def body(r, carry):
                    iw0 = ixs[p].at[r, pl.ds(0, IDX_W)]
                    iw1 = ixs[p].at[r, pl.ds(IDX_W, LTAIL)]
                    d0 = pl.ds(r * L, IDX_W)
                    d1 = pl.ds(r * L + IDX_W, LTAIL)
                    pltpu.async_copy(t_tf.at[iw0], gtfs[p].at[d0], sems[p])
                    pltpu.async_copy(t_tf.at[iw1], gtfs[p].at[d1], sems[p])
                    pltpu.async_copy(t_ti.at[iw0], gtis[p].at[d0], sems[p])
                    pltpu.async_copy(t_ti.at[iw1], gtis[p].at[d1], sems[p])
                    return carry
                lax.fori_loop(0, CH, body, 0)

            def drain(c):
                p = c % 2
                pltpu.make_async_copy(t_tf.at[pl.ds(0, N)], gtfs[p], sems[p]).wait()
                pltpu.make_async_copy(t_ti.at[pl.ds(0, N)], gtis[p], sems[p]).wait()

            def compute(c):
                p = c % 2
                for g in range(CH // LANES):
                    goff = g * LANES

                    def cbody(j, carry):
                        a_tf, a_ti = carry
                        idxv = row_lanes + (goff * L + j)
                        jcol = lane_iota * 0 + j
                        vtf = plsc.load_gather(gtfs[p], [idxv])
                        vti = plsc.load_gather(gtis[p], [idxv])
                        vq = plsc.load_gather(vvs[p], [lane_iota + goff, jcol])
                        return (a_tf + vtf, a_ti + vti * vq)
                    zero = jnp.zeros((LANES,), jnp.float32)
                    a_tf, a_ti = lax.fori_loop(0, L, cbody, (zero, zero),
                                               unroll=4)
                    outv[pl.ds(goff, LANES)] = a_tf * inv_l + a_ti
                dst = pl.ds(out_base + sid * ROWS_T + c * CH, CH)
                pltpu.sync_copy(outv, out_hbm.at[dst])

            stage(0)
            fire(0)
            for c in range(NCH):
                if c + 1 < NCH:
                    stage(c + 1)
                    fire(c + 1)
                drain(c)
                compute(c)

        @pl.when(cid == 0)
        def _():
            pipeline(xi_hbm, xv_hbm, wtf, wti, 0)

        @pl.when(cid == 1)
        def _():
            pipeline(yi_hbm, yv_hbm, utf, uti, B)

    bias = 0.0 + 0.0001 * float(L) / 10.0 + 0.0001 * float(L) / 10.0

    def fin_body(p_ref, o_ref):
        z = p_ref[0, :] + p_ref[1, :] + bias
        o_ref[...] = 1.0 / (1.0 + jnp.exp(-z))

    finisher = pl.pallas_call(
        fin_body,
        out_shape=jax.ShapeDtypeStruct((B,), jnp.float32),
    )

    def run(x_idx, y_idx, x_tfidf, y_tfidf, W_tf, U_tf, W_tfidf, U_tfidf):
        partials = sc_kernel(x_idx, y_idx, x_tfidf, y_tfidf,
                             W_tf, W_tfidf, U_tf, U_tfidf)
        return finisher(partials.reshape(2, B))

    return run


def kernel(x_idx, y_idx, x_tfidf, y_tfidf, W_tf, U_tf, W_tfidf, U_tfidf):
    B, L = x_idx.shape
    V = W_tf.shape[0]
    return _build(B, L, V)(x_idx, y_idx, x_tfidf, y_tfidf,
                           W_tf, U_tf, W_tfidf, U_tfidf)
